# baseline (speedup 1.0000x reference)
import functools

import jax
import jax.numpy as jnp
from jax.experimental import pallas as pl
from jax.experimental.pallas import tpu as pltpu

_HEADS = 8
_NEG = -1e30
_HI = jax.lax.Precision.HIGHEST
_STREAM = jax.lax.Precision.DEFAULT


def _agg_kernel(gids_ref, x_ref, query_ref, wq_ref, bq_ref, wk_ref, bk_ref,
                wv_ref, bv_ref, outw_ref, outb_ref, linw_ref, linb_ref,
                out_ref, m_ref, den_ref, num_ref, *, nsteps, heads, dh,
                ngroups):
    pid = pl.program_id(0)

    @pl.when(pid == 0)
    def _init():
        m_ref[...] = jnp.full_like(m_ref, _NEG)
        den_ref[...] = jnp.zeros_like(den_ref)
        num_ref[...] = jnp.zeros_like(num_ref)

    emb = x_ref.shape[1]
    cols = ngroups * heads
    rr = jax.lax.broadcasted_iota(jnp.int32, (heads, emb), 1) // dh
    hh = jax.lax.broadcasted_iota(jnp.int32, (heads, emb), 0)
    hsel = (rr == hh).astype(jnp.float32)
    cc = jax.lax.broadcasted_iota(jnp.int32, (heads, cols), 1) // ngroups
    h2 = jax.lax.broadcasted_iota(jnp.int32, (heads, cols), 0)
    r2 = (cc == h2).astype(jnp.float32)

    qp = jax.lax.dot_general(wq_ref[...], query_ref[...],
                             (((1,), (0,)), ((), ())),
                             precision=_HI) + bq_ref[...]
    inv = 1.0 / jnp.sqrt(jnp.float32(dh))
    a_mat = jax.lax.dot_general(hsel, qp * wk_ref[...],
                                (((1,), (0,)), ((), ())),
                                precision=_HI) * inv
    c_row = jax.lax.dot_general(qp * bk_ref[...], hsel,
                                (((0,), (1,)), ((), ())),
                                precision=_HI) * inv
    afull = jax.lax.dot_general(r2, a_mat, (((0,), (0,)), ((), ())),
                                precision=_HI)
    cb = jax.lax.dot_general(c_row, r2, (((1,), (0,)), ((), ())),
                             precision=_HI)

    x = x_ref[...]
    sb = jax.lax.dot_general(x, afull, (((1,), (1,)), ((), ())),
                             precision=_STREAM) + cb

    gids = gids_ref[0]
    colg = jax.lax.broadcasted_iota(jnp.int32, (1, cols), 1) % ngroups
    ohm = gids == colg

    bigf = jnp.where(ohm, sb, _NEG)
    m_tile = jnp.max(bigf, axis=0, keepdims=True)
    m_old = m_ref[...]
    m_new = jnp.maximum(m_old, m_tile)
    alpha = jnp.exp(m_old - m_new)
    ef = jnp.exp(jnp.where(ohm, sb - m_new, _NEG))
    den_ref[...] = den_ref[...] * alpha + jnp.sum(ef, axis=0, keepdims=True)
    m_ref[...] = m_new
    num_ref[...] = num_ref[...] * alpha + jax.lax.dot_general(
        x, ef, (((0,), (0,)), ((), ())), precision=_STREAM)

    @pl.when(pid == nsteps - 1)
    def _fin():
        ybar = num_ref[...] / den_ref[...]
        blocks = []
        for h in range(heads):
            yh = ybar[:, h * ngroups:(h + 1) * ngroups]
            wvh = wv_ref[h * dh:(h + 1) * dh, :]
            blocks.append(jax.lax.dot_general(
                yh, wvh, (((0,), (1,)), ((), ())), precision=_HI))
        ctx = jnp.concatenate(blocks, axis=1) + bv_ref[...]
        ge = jax.lax.dot_general(ctx, outw_ref[...], (((1,), (1,)), ((), ())),
                                 precision=_HI) + outb_ref[...]
        out_ref[...] = jax.lax.dot_general(ge, linw_ref[...],
                                           (((1,), (1,)), ((), ())),
                                           precision=_HI) + linb_ref[...]


def kernel(tree_preds, query, in_proj_w, in_proj_b, out_w, out_b, lin_w,
           lin_b, group_ids):
    n, emb = tree_preds.shape
    heads = _HEADS
    dh = emb // heads
    ngroups = 16
    nout = lin_w.shape[0]
    tile = 2048
    nsteps = n // tile

    gids3 = group_ids.astype(jnp.int32).reshape(nsteps, tile, 1)
    wq = in_proj_w[:emb]
    wk = in_proj_w[emb:2 * emb]
    wv = in_proj_w[2 * emb:]
    bq = in_proj_b[:emb].reshape(emb, 1)
    bk = in_proj_b[emb:2 * emb].reshape(emb, 1)
    bv = in_proj_b[2 * emb:].reshape(1, emb)
    queryr = query.reshape(emb, 1)
    outb = out_b.reshape(1, emb)
    linw = jnp.pad(lin_w, ((0, emb - nout), (0, 0)))
    linb = jnp.pad(lin_b, (0, emb - nout)).reshape(1, emb)

    def full(arr):
        return pl.BlockSpec(arr.shape, lambda i: (0,) * arr.ndim)

    res = pl.pallas_call(
        functools.partial(_agg_kernel, nsteps=nsteps, heads=heads, dh=dh,
                          ngroups=ngroups),
        grid=(nsteps,),
        in_specs=[
            pl.BlockSpec((1, tile, 1), lambda i: (i, 0, 0)),
            pl.BlockSpec((tile, emb), lambda i: (i, 0)),
            full(queryr), full(wq), full(bq), full(wk), full(bk),
            full(wv), full(bv), full(out_w), full(outb), full(linw),
            full(linb),
        ],
        out_specs=pl.BlockSpec((ngroups, emb), lambda i: (0, 0)),
        out_shape=jax.ShapeDtypeStruct((ngroups, emb), jnp.float32),
        scratch_shapes=[
            pltpu.VMEM((1, ngroups * heads), jnp.float32),
            pltpu.VMEM((1, ngroups * heads), jnp.float32),
            pltpu.VMEM((emb, ngroups * heads), jnp.float32),
        ],
    )(gids3, tree_preds, queryr, wq, bq, wk, bk, wv, bv, out_w, outb,
      linw, linb)
    return res[:, :nout]


# speedup vs baseline: 1.1938x; 1.1938x over previous
import functools

import jax
import jax.numpy as jnp
from jax.experimental import pallas as pl
from jax.experimental.pallas import tpu as pltpu

_HEADS = 8
_NEG = -1e30
_HI = jax.lax.Precision.HIGHEST
_STREAM = jax.lax.Precision.DEFAULT


def _agg_kernel(gids_ref, x_ref, query_ref, wq_ref, bq_ref, wk_ref, bk_ref,
                wv_ref, bv_ref, outw_ref, outb_ref, linw_ref, linb_ref,
                out_ref, m_ref, den_ref, num_ref, *, nsteps, heads, dh,
                ngroups):
    pid = pl.program_id(0)

    @pl.when(pid == 0)
    def _init():
        m_ref[...] = jnp.full_like(m_ref, _NEG)
        den_ref[...] = jnp.zeros_like(den_ref)
        num_ref[...] = jnp.zeros_like(num_ref)

    emb = x_ref.shape[1]
    cols = ngroups * heads
    rr = jax.lax.broadcasted_iota(jnp.int32, (heads, emb), 1) // dh
    hh = jax.lax.broadcasted_iota(jnp.int32, (heads, emb), 0)
    hsel = (rr == hh).astype(jnp.float32)
    cc = jax.lax.broadcasted_iota(jnp.int32, (heads, cols), 1) // ngroups
    h2 = jax.lax.broadcasted_iota(jnp.int32, (heads, cols), 0)
    r2 = (cc == h2).astype(jnp.float32)

    qp = jax.lax.dot_general(wq_ref[...], query_ref[...],
                             (((1,), (0,)), ((), ())),
                             precision=_HI) + bq_ref[...]
    inv = 1.0 / jnp.sqrt(jnp.float32(dh))
    a_mat = jax.lax.dot_general(hsel, qp * wk_ref[...],
                                (((1,), (0,)), ((), ())),
                                precision=_HI) * inv
    c_row = jax.lax.dot_general(qp * bk_ref[...], hsel,
                                (((0,), (1,)), ((), ())),
                                precision=_HI) * inv
    afull = jax.lax.dot_general(r2, a_mat, (((0,), (0,)), ((), ())),
                                precision=_HI)
    cb = jax.lax.dot_general(c_row, r2, (((1,), (0,)), ((), ())),
                             precision=_HI)

    x = x_ref[...]
    sb = jax.lax.dot_general(x, afull, (((1,), (1,)), ((), ())),
                             precision=_STREAM) + cb

    gids = gids_ref[0]
    colg = jax.lax.broadcasted_iota(jnp.int32, (1, cols), 1) % ngroups
    ohm = gids == colg

    bigf = jnp.where(ohm, sb, _NEG)
    m_tile = jnp.max(bigf, axis=0, keepdims=True)
    m_old = m_ref[...]
    m_new = jnp.maximum(m_old, m_tile)
    alpha = jnp.exp(m_old - m_new)
    ef = jnp.exp(jnp.where(ohm, sb - m_new, _NEG))
    den_ref[...] = den_ref[...] * alpha + jnp.sum(ef, axis=0, keepdims=True)
    m_ref[...] = m_new
    num_ref[...] = num_ref[...] * alpha + jax.lax.dot_general(
        x, ef, (((0,), (0,)), ((), ())), precision=_STREAM)

    @pl.when(pid == nsteps - 1)
    def _fin():
        ybar = num_ref[...] / den_ref[...]
        blocks = []
        for h in range(heads):
            yh = ybar[:, h * ngroups:(h + 1) * ngroups]
            wvh = wv_ref[h * dh:(h + 1) * dh, :]
            blocks.append(jax.lax.dot_general(
                yh, wvh, (((0,), (1,)), ((), ())), precision=_HI))
        ctx = jnp.concatenate(blocks, axis=1) + bv_ref[...]
        ge = jax.lax.dot_general(ctx, outw_ref[...], (((1,), (1,)), ((), ())),
                                 precision=_HI) + outb_ref[...]
        out_ref[...] = jax.lax.dot_general(ge, linw_ref[...],
                                           (((1,), (1,)), ((), ())),
                                           precision=_HI) + linb_ref[...]


def kernel(tree_preds, query, in_proj_w, in_proj_b, out_w, out_b, lin_w,
           lin_b, group_ids):
    n, emb = tree_preds.shape
    heads = _HEADS
    dh = emb // heads
    ngroups = 16
    nout = lin_w.shape[0]
    tile = 16384
    nsteps = n // tile

    gids3 = group_ids.astype(jnp.int32).reshape(nsteps, tile, 1)
    wq = in_proj_w[:emb]
    wk = in_proj_w[emb:2 * emb]
    wv = in_proj_w[2 * emb:]
    bq = in_proj_b[:emb].reshape(emb, 1)
    bk = in_proj_b[emb:2 * emb].reshape(emb, 1)
    bv = in_proj_b[2 * emb:].reshape(1, emb)
    queryr = query.reshape(emb, 1)
    outb = out_b.reshape(1, emb)
    linw = jnp.pad(lin_w, ((0, emb - nout), (0, 0)))
    linb = jnp.pad(lin_b, (0, emb - nout)).reshape(1, emb)

    def full(arr):
        return pl.BlockSpec(arr.shape, lambda i: (0,) * arr.ndim)

    res = pl.pallas_call(
        functools.partial(_agg_kernel, nsteps=nsteps, heads=heads, dh=dh,
                          ngroups=ngroups),
        grid=(nsteps,),
        in_specs=[
            pl.BlockSpec((1, tile, 1), lambda i: (i, 0, 0)),
            pl.BlockSpec((tile, emb), lambda i: (i, 0)),
            full(queryr), full(wq), full(bq), full(wk), full(bk),
            full(wv), full(bv), full(out_w), full(outb), full(linw),
            full(linb),
        ],
        out_specs=pl.BlockSpec((ngroups, emb), lambda i: (0, 0)),
        out_shape=jax.ShapeDtypeStruct((ngroups, emb), jnp.float32),
        scratch_shapes=[
            pltpu.VMEM((1, ngroups * heads), jnp.float32),
            pltpu.VMEM((1, ngroups * heads), jnp.float32),
            pltpu.VMEM((emb, ngroups * heads), jnp.float32),
        ],
    )(gids3, tree_preds, queryr, wq, bq, wk, bk, wv, bv, out_w, outb,
      linw, linb)
    return res[:, :nout]


# speedup vs baseline: 1.1960x; 1.0019x over previous
import functools

import jax
import jax.numpy as jnp
from jax.experimental import pallas as pl
from jax.experimental.pallas import tpu as pltpu

_HEADS = 8
_NEG = -1e30
_HI = jax.lax.Precision.HIGHEST
_STREAM = jax.lax.Precision.DEFAULT


def _agg_kernel(gids_ref, x_ref, query_ref, wq_ref, bq_ref, wk_ref, bk_ref,
                wv_ref, bv_ref, outw_ref, outb_ref, linw_ref, linb_ref,
                out_ref, m_ref, den_ref, num_ref, *, nsteps, heads, dh,
                ngroups):
    pid = pl.program_id(0)

    @pl.when(pid == 0)
    def _init():
        m_ref[...] = jnp.full_like(m_ref, _NEG)
        den_ref[...] = jnp.zeros_like(den_ref)
        num_ref[...] = jnp.zeros_like(num_ref)

    emb = x_ref.shape[1]
    cols = ngroups * heads
    rr = jax.lax.broadcasted_iota(jnp.int32, (heads, emb), 1) // dh
    hh = jax.lax.broadcasted_iota(jnp.int32, (heads, emb), 0)
    hsel = (rr == hh).astype(jnp.float32)
    cc = jax.lax.broadcasted_iota(jnp.int32, (heads, cols), 1) // ngroups
    h2 = jax.lax.broadcasted_iota(jnp.int32, (heads, cols), 0)
    r2 = (cc == h2).astype(jnp.float32)

    qp = jax.lax.dot_general(wq_ref[...], query_ref[...],
                             (((1,), (0,)), ((), ())),
                             precision=_HI) + bq_ref[...]
    inv = 1.0 / jnp.sqrt(jnp.float32(dh))
    a_mat = jax.lax.dot_general(hsel, qp * wk_ref[...],
                                (((1,), (0,)), ((), ())),
                                precision=_HI) * inv
    c_row = jax.lax.dot_general(qp * bk_ref[...], hsel,
                                (((0,), (1,)), ((), ())),
                                precision=_HI) * inv
    afull = jax.lax.dot_general(r2, a_mat, (((0,), (0,)), ((), ())),
                                precision=_HI)
    cb = jax.lax.dot_general(c_row, r2, (((1,), (0,)), ((), ())),
                             precision=_HI)

    x = x_ref[...]
    sb = jax.lax.dot_general(x, afull, (((1,), (1,)), ((), ())),
                             precision=_STREAM) + cb

    gids = gids_ref[0]
    colg = jax.lax.broadcasted_iota(jnp.int32, (1, cols), 1) % ngroups
    ohm = gids == colg

    bigf = jnp.where(ohm, sb, _NEG)
    m_tile = jnp.max(bigf, axis=0, keepdims=True)
    m_old = m_ref[...]
    m_new = jnp.maximum(m_old, m_tile)
    alpha = jnp.exp(m_old - m_new)
    ef = jnp.exp(bigf - m_new)
    ones_row = jnp.ones((1, x.shape[0]), jnp.float32)
    den_tile = jax.lax.dot_general(ones_row, ef, (((1,), (0,)), ((), ())),
                                   precision=_STREAM)
    den_ref[...] = den_ref[...] * alpha + den_tile
    m_ref[...] = m_new
    num_ref[...] = num_ref[...] * alpha + jax.lax.dot_general(
        x, ef, (((0,), (0,)), ((), ())), precision=_STREAM)

    @pl.when(pid == nsteps - 1)
    def _fin():
        ybar = num_ref[...] / den_ref[...]
        blocks = []
        for h in range(heads):
            yh = ybar[:, h * ngroups:(h + 1) * ngroups]
            wvh = wv_ref[h * dh:(h + 1) * dh, :]
            blocks.append(jax.lax.dot_general(
                yh, wvh, (((0,), (1,)), ((), ())), precision=_HI))
        ctx = jnp.concatenate(blocks, axis=1) + bv_ref[...]
        ge = jax.lax.dot_general(ctx, outw_ref[...], (((1,), (1,)), ((), ())),
                                 precision=_HI) + outb_ref[...]
        out_ref[...] = jax.lax.dot_general(ge, linw_ref[...],
                                           (((1,), (1,)), ((), ())),
                                           precision=_HI) + linb_ref[...]


def kernel(tree_preds, query, in_proj_w, in_proj_b, out_w, out_b, lin_w,
           lin_b, group_ids):
    n, emb = tree_preds.shape
    heads = _HEADS
    dh = emb // heads
    ngroups = 16
    nout = lin_w.shape[0]
    tile = 8192
    nsteps = n // tile

    gids3 = group_ids.astype(jnp.int32).reshape(nsteps, tile, 1)
    wq = in_proj_w[:emb]
    wk = in_proj_w[emb:2 * emb]
    wv = in_proj_w[2 * emb:]
    bq = in_proj_b[:emb].reshape(emb, 1)
    bk = in_proj_b[emb:2 * emb].reshape(emb, 1)
    bv = in_proj_b[2 * emb:].reshape(1, emb)
    queryr = query.reshape(emb, 1)
    outb = out_b.reshape(1, emb)
    linw = jnp.pad(lin_w, ((0, emb - nout), (0, 0)))
    linb = jnp.pad(lin_b, (0, emb - nout)).reshape(1, emb)

    def full(arr):
        return pl.BlockSpec(arr.shape, lambda i: (0,) * arr.ndim)

    res = pl.pallas_call(
        functools.partial(_agg_kernel, nsteps=nsteps, heads=heads, dh=dh,
                          ngroups=ngroups),
        grid=(nsteps,),
        in_specs=[
            pl.BlockSpec((1, tile, 1), lambda i: (i, 0, 0)),
            pl.BlockSpec((tile, emb), lambda i: (i, 0)),
            full(queryr), full(wq), full(bq), full(wk), full(bk),
            full(wv), full(bv), full(out_w), full(outb), full(linw),
            full(linb),
        ],
        out_specs=pl.BlockSpec((ngroups, emb), lambda i: (0, 0)),
        out_shape=jax.ShapeDtypeStruct((ngroups, emb), jnp.float32),
        scratch_shapes=[
            pltpu.VMEM((1, ngroups * heads), jnp.float32),
            pltpu.VMEM((1, ngroups * heads), jnp.float32),
            pltpu.VMEM((emb, ngroups * heads), jnp.float32),
        ],
    )(gids3, tree_preds, queryr, wq, bq, wk, bk, wv, bv, out_w, outb,
      linw, linb)
    return res[:, :nout]


# speedup vs baseline: 1.3724x; 1.1475x over previous
import functools

import jax
import jax.numpy as jnp
from jax.experimental import pallas as pl
from jax.experimental.pallas import tpu as pltpu

_HEADS = 8
_NEG = -1e30
_HI = jax.lax.Precision.HIGHEST
_STREAM = jax.lax.Precision.DEFAULT


def _agg_kernel(gids_ref, x_ref, query_ref, inw_ref, inb_ref, outw_ref,
                outb_ref, linw_ref, out_ref, m_ref, den_ref, num_ref, *,
                nsteps, heads, dh, ngroups):
    pid = pl.program_id(0)

    @pl.when(pid == 0)
    def _init():
        m_ref[...] = jnp.full_like(m_ref, _NEG)
        den_ref[...] = jnp.zeros_like(den_ref)
        num_ref[...] = jnp.zeros_like(num_ref)

    emb = x_ref.shape[1]
    cols = ngroups * heads
    rr = jax.lax.broadcasted_iota(jnp.int32, (heads, emb), 1) // dh
    hh = jax.lax.broadcasted_iota(jnp.int32, (heads, emb), 0)
    hsel = (rr == hh).astype(jnp.float32)
    cc = jax.lax.broadcasted_iota(jnp.int32, (heads, cols), 1) // ngroups
    h2 = jax.lax.broadcasted_iota(jnp.int32, (heads, cols), 0)
    r2 = (cc == h2).astype(jnp.float32)
    ident = (jax.lax.broadcasted_iota(jnp.int32, (emb, emb), 0) ==
             jax.lax.broadcasted_iota(jnp.int32, (emb, emb), 1)
             ).astype(jnp.float32)

    wq = inw_ref[0:emb, :]
    wk = inw_ref[emb:2 * emb, :]
    sel0 = (jax.lax.broadcasted_iota(jnp.int32, (1, 3), 1) == 0
            ).astype(jnp.float32)
    sel1 = (jax.lax.broadcasted_iota(jnp.int32, (1, 3), 1) == 1
            ).astype(jnp.float32)
    inb = inb_ref[...]
    bq_row = jax.lax.dot_general(sel0, inb, (((1,), (0,)), ((), ())),
                                 precision=_HI)
    bk_row = jax.lax.dot_general(sel1, inb, (((1,), (0,)), ((), ())),
                                 precision=_HI)

    qrow = query_ref[0]
    qp_row = jax.lax.dot_general(qrow, wq, (((1,), (1,)), ((), ())),
                                 precision=_HI) + bq_row
    qp_col = jax.lax.dot_general(ident, qp_row, (((1,), (1,)), ((), ())),
                                 precision=_HI)
    inv = 1.0 / jnp.sqrt(jnp.float32(dh))
    a_mat = jax.lax.dot_general(hsel, qp_col * wk,
                                (((1,), (0,)), ((), ())),
                                precision=_HI) * inv
    c_row = jax.lax.dot_general(qp_row * bk_row, hsel,
                                (((1,), (1,)), ((), ())),
                                precision=_HI) * inv
    afull = jax.lax.dot_general(r2, a_mat, (((0,), (0,)), ((), ())),
                                precision=_HI)
    cb = jax.lax.dot_general(c_row, r2, (((1,), (0,)), ((), ())),
                             precision=_HI)

    x = x_ref[...]
    sb = jax.lax.dot_general(x, afull, (((1,), (1,)), ((), ())),
                             precision=_STREAM) + cb

    gids = gids_ref[0]
    colg = jax.lax.broadcasted_iota(jnp.int32, (1, cols), 1) % ngroups
    ohm = gids == colg

    bigf = jnp.where(ohm, sb, _NEG)
    m_tile = jnp.max(bigf, axis=0, keepdims=True)
    m_old = m_ref[...]
    m_new = jnp.maximum(m_old, m_tile)
    alpha = jnp.exp(m_old - m_new)
    ef = jnp.exp(bigf - m_new)
    ones_row = jnp.ones((1, x.shape[0]), jnp.float32)
    den_tile = jax.lax.dot_general(ones_row, ef, (((1,), (0,)), ((), ())),
                                   precision=_STREAM)
    den_ref[...] = den_ref[...] * alpha + den_tile
    m_ref[...] = m_new
    num_ref[...] = num_ref[...] * alpha + jax.lax.dot_general(
        x, ef, (((0,), (0,)), ((), ())), precision=_STREAM)

    @pl.when(pid == nsteps - 1)
    def _fin():
        wv = inw_ref[2 * emb:3 * emb, :]
        sel2 = (jax.lax.broadcasted_iota(jnp.int32, (1, 3), 1) == 2
                ).astype(jnp.float32)
        bv_row = jax.lax.dot_general(sel2, inb_ref[...],
                                     (((1,), (0,)), ((), ())),
                                     precision=_HI)
        ybar = num_ref[...] / den_ref[...]
        blocks = []
        for h in range(heads):
            yh = ybar[:, h * ngroups:(h + 1) * ngroups]
            wvh = wv[h * dh:(h + 1) * dh, :]
            blocks.append(jax.lax.dot_general(
                yh, wvh, (((0,), (1,)), ((), ())), precision=_HI))
        ctx = jnp.concatenate(blocks, axis=1) + bv_row
        ge = jax.lax.dot_general(ctx, outw_ref[...], (((1,), (1,)), ((), ())),
                                 precision=_HI) + outb_ref[...]
        e0_col = (jax.lax.broadcasted_iota(jnp.int32, (emb, 1), 0) == 0
                  ).astype(jnp.float32)
        linw_pad = jax.lax.dot_general(e0_col, linw_ref[...],
                                       (((1,), (0,)), ((), ())),
                                       precision=_HI)
        out_ref[...] = jax.lax.dot_general(ge, linw_pad,
                                           (((1,), (1,)), ((), ())),
                                           precision=_HI)


def kernel(tree_preds, query, in_proj_w, in_proj_b, out_w, out_b, lin_w,
           lin_b, group_ids):
    n, emb = tree_preds.shape
    heads = _HEADS
    dh = emb // heads
    ngroups = 16
    nout = lin_w.shape[0]
    tile = 8192
    nsteps = n // tile

    gids3 = group_ids.astype(jnp.int32).reshape(nsteps, tile, 1)
    inb = in_proj_b.reshape(3, emb)
    outb = out_b.reshape(1, emb)

    def full(arr):
        return pl.BlockSpec(arr.shape, lambda i: (0,) * arr.ndim)

    res = pl.pallas_call(
        functools.partial(_agg_kernel, nsteps=nsteps, heads=heads, dh=dh,
                          ngroups=ngroups),
        grid=(nsteps,),
        in_specs=[
            pl.BlockSpec((1, tile, 1), lambda i: (i, 0, 0)),
            pl.BlockSpec((tile, emb), lambda i: (i, 0)),
            full(query), full(in_proj_w), full(inb), full(out_w),
            full(outb), full(lin_w),
        ],
        out_specs=pl.BlockSpec((ngroups, emb), lambda i: (0, 0)),
        out_shape=jax.ShapeDtypeStruct((ngroups, emb), jnp.float32),
        scratch_shapes=[
            pltpu.VMEM((1, ngroups * heads), jnp.float32),
            pltpu.VMEM((1, ngroups * heads), jnp.float32),
            pltpu.VMEM((emb, ngroups * heads), jnp.float32),
        ],
    )(gids3, tree_preds, query, in_proj_w, inb, out_w, outb, lin_w)
    return res[:, :nout] + lin_b
